# Initial kernel scaffold; baseline (speedup 1.0000x reference)
import jax
import jax.numpy as jnp
from jax.experimental import pallas as pl


def kernel(activations, state, probe_candidates, selection_logits, steering_dirs, W1, b1, W2, b2, scale):
    raise NotImplementedError("write your pallas kernel here")


# speedup vs baseline: 226.6353x; 226.6353x over previous
import functools

import jax
import jax.numpy as jnp
from jax import lax
from jax.experimental import pallas as pl
from jax.experimental.pallas import tpu as pltpu
from jax.experimental.pallas import tpu_sc as plsc

_N_FEAT = 131072
_N_PROBES = 8192
_N_CAND = 512
_D_MODEL = 4096

_NC = 2
_NS = 16
_L = 16
_NW = _NC * _NS
_P_PER_W = _N_PROBES // _NW
_CHUNK = 16
_N_CHUNKS = _P_PER_W // _CHUNK
_G = _N_CAND // _L


def _probe_values_sc(packed_table, probe_candidates, selection_logits):
  mesh = plsc.VectorSubcoreMesh(core_axis_name="c", subcore_axis_name="s")

  @functools.partial(
      pl.kernel,
      mesh=mesh,
      out_type=jax.ShapeDtypeStruct((_N_PROBES,), jnp.float32),
      compiler_params=pltpu.CompilerParams(needs_layout_passes=False),
      scratch_types=[
          pltpu.VMEM((_N_FEAT // 2,), jnp.int32),
          pltpu.VMEM((_CHUNK, _N_CAND), jnp.int32),
          pltpu.VMEM((_CHUNK, _N_CAND), jnp.float32),
          pltpu.VMEM((_CHUNK,), jnp.float32),
      ],
  )
  def body(table_hbm, idx_hbm, logit_hbm, out_hbm, table_v, idx_v, log_v, pv_v):
    wid = lax.axis_index("s") * _NC + lax.axis_index("c")
    base = wid * _P_PER_W
    pltpu.sync_copy(table_hbm, table_v)
    lane = lax.broadcasted_iota(jnp.int32, (_L,), 0)

    def chunk_body(c, carry):
      row0 = base + c * _CHUNK
      pltpu.sync_copy(idx_hbm.at[pl.ds(row0, _CHUNK), :], idx_v)
      pltpu.sync_copy(logit_hbm.at[pl.ds(row0, _CHUNK), :], log_v)

      def probe_body(p, carry2):
        acc = jnp.zeros((_L,), jnp.float32)
        wsum = jnp.zeros((_L,), jnp.float32)
        for g in range(_G):
          lg = log_v[p, pl.ds(g * _L, _L)]
          e = jnp.exp(lg)
          iv = idx_v[p, pl.ds(g * _L, _L)]
          widx = lax.shift_right_logical(iv, 1)
          wbits = plsc.load_gather(table_v, [widx])
          odd = lax.bitwise_and(iv, 1) == 1
          bits = jnp.where(odd, wbits, lax.shift_left(wbits, 16))
          bits = lax.bitwise_and(bits, jnp.int32(-65536))
          val = lax.bitcast_convert_type(bits, jnp.float32)
          acc = acc + e * val
          wsum = wsum + e
        num = jnp.broadcast_to(jnp.sum(acc), (_L,))
        den = jnp.broadcast_to(jnp.sum(wsum), (_L,))
        plsc.store_scatter(
            pv_v,
            [jnp.broadcast_to(p, (_L,)).astype(jnp.int32)],
            num / den,
            mask=lane == 0,
        )
        return carry2

      lax.fori_loop(0, _CHUNK, probe_body, 0)
      pltpu.sync_copy(pv_v, out_hbm.at[pl.ds(row0, _CHUNK)])
      return carry

    lax.fori_loop(0, _N_CHUNKS, chunk_body, 0)

  return body(packed_table, probe_candidates, selection_logits)


_PB = 512
_NB = _N_PROBES // _PB


def _steer_tc(pv, state, W1, b1, W2, b2, steering_dirs):
  def body(state_ref, w1_ref, b1_ref, pv_ref, w2_ref, b2_ref, dirs_ref,
           out_ref, acc_ref):
    i = pl.program_id(0)
    st = state_ref[...]
    z = jnp.sum(w1_ref[...] * st, axis=1) + b1_ref[0, :]
    h = 0.5 * z * (1.0 + lax.erf(z * jnp.float32(0.7071067811865476)))
    m = jnp.sum(w2_ref[...] * h[None, :], axis=1) + b2_ref[0, :]
    wvec = pv_ref[0, :] * jax.nn.sigmoid(m)
    contrib = jnp.dot(wvec[None, :], dirs_ref[...],
                      preferred_element_type=jnp.float32)

    @pl.when(i == 0)
    def _():
      acc_ref[...] = jnp.zeros_like(acc_ref)

    acc_ref[...] += contrib

    @pl.when(i == _NB - 1)
    def _():
      out_ref[...] = jnp.tanh(acc_ref[...])

  return pl.pallas_call(
      body,
      grid=(_NB,),
      in_specs=[
          pl.BlockSpec((1, 4), lambda i: (0, 0)),
          pl.BlockSpec((32, 4), lambda i: (0, 0)),
          pl.BlockSpec((1, 32), lambda i: (0, 0)),
          pl.BlockSpec((1, _PB), lambda i: (0, i)),
          pl.BlockSpec((_PB, 32), lambda i: (i, 0)),
          pl.BlockSpec((1, _PB), lambda i: (0, i)),
          pl.BlockSpec((_PB, _D_MODEL), lambda i: (i, 0)),
      ],
      out_specs=pl.BlockSpec((1, _D_MODEL), lambda i: (0, 0)),
      out_shape=jax.ShapeDtypeStruct((1, _D_MODEL), jnp.float32),
      scratch_shapes=[pltpu.VMEM((1, _D_MODEL), jnp.float32)],
      compiler_params=pltpu.CompilerParams(
          dimension_semantics=("arbitrary",)),
  )(state.reshape(1, 4), W1, b1.reshape(1, 32), pv.reshape(1, _N_PROBES),
    W2, b2.reshape(1, _N_PROBES), steering_dirs)


def kernel(activations, state, probe_candidates, selection_logits,
           steering_dirs, W1, b1, W2, b2, scale):
  acts_bf = activations.astype(jnp.bfloat16)
  packed = lax.bitcast_convert_type(
      acts_bf.reshape(_N_FEAT // 2, 2), jnp.int32)
  pv = _probe_values_sc(packed, probe_candidates, selection_logits)
  steer = _steer_tc(pv, state, W1, b1, W2, b2, steering_dirs)
  return steer.reshape(_D_MODEL) * (scale * 10.0)
